# baseline (speedup 1.0000x reference)
import jax
import jax.numpy as jnp
from jax import lax
from jax.experimental import pallas as pl
from jax.experimental.pallas import tpu as pltpu
from jax.experimental.pallas import tpu_sc as plsc
import functools

NUM_RADIAL = 8
NUM_SPHERICAL = 8
L = 16
NC = 2
NS = 16
NW = NC * NS

CHUNK = 640
GROUPS = CHUNK // L
IDX_ROWS = CHUNK // 128

R_OFFS = [r / (NUM_RADIAL - 1) for r in range(NUM_RADIAL)]
R_COEFF = -0.5 * (NUM_RADIAL - 1) ** 2
S_OFFS = [-1.0 + 2.0 * s / (NUM_SPHERICAL - 1) for s in range(NUM_SPHERICAL)]
S_COEFF = -0.5 * ((NUM_SPHERICAL - 1) / 2.0) ** 2


def _make_kernel(T):
    n_chunks = T // CHUNK
    mesh = plsc.VectorSubcoreMesh(
        core_axis_name="c", subcore_axis_name="s",
        num_cores=NC, num_subcores=NS)

    @functools.partial(
        pl.kernel,
        out_type=jax.ShapeDtypeStruct((T * 64,), jnp.float32),
        mesh=mesh,
        compiler_params=pltpu.CompilerParams(use_tc_tiling_on_sc=False,
                                             needs_layout_passes=False),
        scratch_types=[
            pltpu.VMEM((IDX_ROWS, 128), jnp.int32),
            pltpu.VMEM((CHUNK,), jnp.float32),
            pltpu.VMEM((CHUNK,), jnp.float32),
            pltpu.VMEM((CHUNK * 64,), jnp.float32),
            pltpu.SemaphoreType.DMA,
        ],
    )
    def sc_kernel(d_hbm, cos_hbm, id3_hbm, out_hbm, idx_v, dg_v, cos_v,
                  out_v, sem):
        wid = lax.axis_index("s") * NC + lax.axis_index("c")
        iota = lax.iota(jnp.int32, L)
        row_base = iota * 64
        nj = (n_chunks - wid + NW - 1) // NW

        @pl.loop(0, nj)
        def _chunk(j):
            cid = wid + j * NW
            base = cid * CHUNK
            pltpu.sync_copy(id3_hbm.at[pl.ds(cid * IDX_ROWS, IDX_ROWS)],
                            idx_v)
            pltpu.sync_copy(cos_hbm.at[pl.ds(base, CHUNK)], cos_v)
            cps = [
                pltpu.async_copy(d_hbm.at[idx_v.at[k]],
                                 dg_v.at[pl.ds(k * 128, 128)], sem)
                for k in range(IDX_ROWS)
            ]
            for cp in cps:
                cp.wait()

            @pl.loop(0, GROUPS)
            def _group(g):
                d = dg_v[pl.ds(g * L, L)]
                c = cos_v[pl.ds(g * L, L)]
                rbf = []
                for r in range(NUM_RADIAL):
                    diff = d - R_OFFS[r]
                    rbf.append(jnp.exp(R_COEFF * diff * diff))
                cbf = []
                for s in range(NUM_SPHERICAL):
                    diff = c - S_OFFS[s]
                    cbf.append(jnp.exp(S_COEFF * diff * diff))
                gbase = row_base + g * (L * 64)
                for s in range(NUM_SPHERICAL):
                    for r in range(NUM_RADIAL):
                        col = s * NUM_RADIAL + r
                        plsc.store_scatter(out_v, [gbase + col],
                                           cbf[s] * rbf[r])

            pltpu.sync_copy(out_v, out_hbm.at[pl.ds(base * 64, CHUNK * 64)])

    return sc_kernel


def kernel(D_ca, cosphi_cab, id3_ca):
    T = cosphi_cab.shape[0]
    id3_2d = jnp.asarray(id3_ca, jnp.int32).reshape(T // 128, 128)
    out_flat = _make_kernel(T)(jnp.asarray(D_ca, jnp.float32),
                               jnp.asarray(cosphi_cab, jnp.float32),
                               id3_2d)
    return (out_flat.reshape(T, NUM_RADIAL * NUM_SPHERICAL),)


# speedup vs baseline: 1.1001x; 1.1001x over previous
import jax
import jax.numpy as jnp
from jax import lax
from jax.experimental import pallas as pl
from jax.experimental.pallas import tpu as pltpu
from jax.experimental.pallas import tpu_sc as plsc
import functools

NUM_RADIAL = 8
NUM_SPHERICAL = 8
L = 16
NC = 2
NS = 16
NW = NC * NS

CHUNK = 640
GROUPS = CHUNK // L
IDX_ROWS = CHUNK // 128
OUT_W = CHUNK * 64

R_OFFS = [r / (NUM_RADIAL - 1) for r in range(NUM_RADIAL)]
R_COEFF = -0.5 * (NUM_RADIAL - 1) ** 2
S_OFFS = [-1.0 + 2.0 * s / (NUM_SPHERICAL - 1) for s in range(NUM_SPHERICAL)]
S_COEFF = -0.5 * ((NUM_SPHERICAL - 1) / 2.0) ** 2


def _make_kernel(T):
    n_chunks = T // CHUNK
    mesh = plsc.VectorSubcoreMesh(
        core_axis_name="c", subcore_axis_name="s",
        num_cores=NC, num_subcores=NS)

    @functools.partial(
        pl.kernel,
        out_type=jax.ShapeDtypeStruct((T * 64,), jnp.float32),
        mesh=mesh,
        compiler_params=pltpu.CompilerParams(use_tc_tiling_on_sc=False,
                                             needs_layout_passes=False),
        scratch_types=[
            pltpu.VMEM((2 * IDX_ROWS, 128), jnp.int32),
            pltpu.VMEM((2 * CHUNK,), jnp.float32),
            pltpu.VMEM((2 * CHUNK,), jnp.float32),
            pltpu.VMEM((2 * OUT_W,), jnp.float32),
            pltpu.SemaphoreType.DMA,
            pltpu.SemaphoreType.DMA,
            pltpu.SemaphoreType.DMA,
            pltpu.SemaphoreType.DMA,
        ],
    )
    def sc_kernel(d_hbm, cos_hbm, id3_hbm, out_hbm, idx_v, dg_v, cos_v,
                  out_v, sem_i, sem_c, sem_g, sem_o):
        wid = lax.axis_index("s") * NC + lax.axis_index("c")
        iota = lax.iota(jnp.int32, L)
        row_base = iota * 64
        nj = (n_chunks - wid + NW - 1) // NW

        def cid_of(j):
            return wid + j * NW

        def issue_in(j, b):
            cid = cid_of(j)
            pltpu.async_copy(id3_hbm.at[pl.ds(cid * IDX_ROWS, IDX_ROWS)],
                             idx_v.at[pl.ds(b * IDX_ROWS, IDX_ROWS)], sem_i)
            pltpu.async_copy(cos_hbm.at[pl.ds(cid * CHUNK, CHUNK)],
                             cos_v.at[pl.ds(b * CHUNK, CHUNK)], sem_c)

        def issue_gather(b):
            for k in range(IDX_ROWS):
                pltpu.async_copy(
                    d_hbm.at[idx_v.at[b * IDX_ROWS + k]],
                    dg_v.at[pl.ds(b * CHUNK + k * 128, 128)], sem_g)

        def wait_gather(b):
            for k in range(IDX_ROWS):
                pltpu.make_async_copy(
                    d_hbm.at[idx_v.at[b * IDX_ROWS + k]],
                    dg_v.at[pl.ds(b * CHUNK + k * 128, 128)], sem_g).wait()

        def out_desc(j, b):
            return pltpu.make_async_copy(
                out_v.at[pl.ds(b * OUT_W, OUT_W)],
                out_hbm.at[pl.ds(cid_of(j) * OUT_W, OUT_W)], sem_o)

        issue_in(0, 0)
        pltpu.make_async_copy(id3_hbm.at[pl.ds(0, IDX_ROWS)],
                              idx_v.at[pl.ds(0, IDX_ROWS)], sem_i).wait()
        issue_gather(0)
        issue_in(1, 1)

        @pl.loop(0, nj)
        def _chunk(j):
            b = lax.rem(j, 2)
            nb = 1 - b

            @pl.when(j + 1 < nj)
            def _():
                pltpu.make_async_copy(
                    id3_hbm.at[pl.ds(0, IDX_ROWS)],
                    idx_v.at[pl.ds(nb * IDX_ROWS, IDX_ROWS)], sem_i).wait()
                issue_gather(nb)

            wait_gather(b)
            pltpu.make_async_copy(cos_hbm.at[pl.ds(0, CHUNK)],
                                  cos_v.at[pl.ds(b * CHUNK, CHUNK)],
                                  sem_c).wait()

            @pl.when(j >= 2)
            def _():
                out_desc(j - 2, b).wait()

            @pl.loop(0, GROUPS)
            def _group(g):
                d = dg_v[pl.ds(b * CHUNK + g * L, L)]
                c = cos_v[pl.ds(b * CHUNK + g * L, L)]
                rbf = []
                for r in range(NUM_RADIAL):
                    diff = d - R_OFFS[r]
                    rbf.append(jnp.exp(R_COEFF * diff * diff))
                cbf = []
                for s in range(NUM_SPHERICAL):
                    diff = c - S_OFFS[s]
                    cbf.append(jnp.exp(S_COEFF * diff * diff))
                gbase = row_base + (b * OUT_W + g * (L * 64))
                for s in range(NUM_SPHERICAL):
                    for r in range(NUM_RADIAL):
                        col = s * NUM_RADIAL + r
                        plsc.store_scatter(out_v, [gbase + col],
                                           cbf[s] * rbf[r])

            pltpu.async_copy(out_v.at[pl.ds(b * OUT_W, OUT_W)],
                             out_hbm.at[pl.ds(cid_of(j) * OUT_W, OUT_W)],
                             sem_o)

            @pl.when(j + 2 < nj)
            def _():
                issue_in(j + 2, b)

        out_desc(nj - 2, lax.rem(nj - 2, 2)).wait()
        out_desc(nj - 1, lax.rem(nj - 1, 2)).wait()

    return sc_kernel


def kernel(D_ca, cosphi_cab, id3_ca):
    T = cosphi_cab.shape[0]
    id3_2d = jnp.asarray(id3_ca, jnp.int32).reshape(T // 128, 128)
    out_flat = _make_kernel(T)(jnp.asarray(D_ca, jnp.float32),
                               jnp.asarray(cosphi_cab, jnp.float32),
                               id3_2d)
    return (out_flat.reshape(T, NUM_RADIAL * NUM_SPHERICAL),)


# speedup vs baseline: 1.7543x; 1.5947x over previous
import jax
import jax.numpy as jnp
from jax import lax
from jax.experimental import pallas as pl
from jax.experimental.pallas import tpu as pltpu
from jax.experimental.pallas import tpu_sc as plsc
import functools

NUM_RADIAL = 8
NUM_SPHERICAL = 8
NC = 2
NS = 16
NW = NC * NS

CHUNK = 640
IDX_ROWS = CHUNK // 128

R_COEFF = -0.5 * (NUM_RADIAL - 1) ** 2
S_COEFF = -0.5 * ((NUM_SPHERICAL - 1) / 2.0) ** 2


def _make_sc_gather(T):
    n_chunks = T // CHUNK
    mesh = plsc.VectorSubcoreMesh(
        core_axis_name="c", subcore_axis_name="s",
        num_cores=NC, num_subcores=NS)

    @functools.partial(
        pl.kernel,
        out_type=jax.ShapeDtypeStruct((T,), jnp.float32),
        mesh=mesh,
        compiler_params=pltpu.CompilerParams(use_tc_tiling_on_sc=False,
                                             needs_layout_passes=False),
        scratch_types=[
            pltpu.VMEM((2 * IDX_ROWS, 128), jnp.int32),
            pltpu.VMEM((2 * CHUNK,), jnp.float32),
            pltpu.SemaphoreType.DMA,
            pltpu.SemaphoreType.DMA,
            pltpu.SemaphoreType.DMA,
        ],
    )
    def sc_kernel(d_hbm, id3_hbm, out_hbm, idx_v, dg_v, sem_i, sem_g, sem_o):
        wid = lax.axis_index("s") * NC + lax.axis_index("c")
        nj = (n_chunks - wid + NW - 1) // NW

        def cid_of(j):
            return wid + j * NW

        def issue_in(j, b):
            pltpu.async_copy(
                id3_hbm.at[pl.ds(cid_of(j) * IDX_ROWS, IDX_ROWS)],
                idx_v.at[pl.ds(b * IDX_ROWS, IDX_ROWS)], sem_i)

        def wait_in(b):
            pltpu.make_async_copy(
                id3_hbm.at[pl.ds(0, IDX_ROWS)],
                idx_v.at[pl.ds(b * IDX_ROWS, IDX_ROWS)], sem_i).wait()

        def issue_gather(b):
            for k in range(IDX_ROWS):
                pltpu.async_copy(
                    d_hbm.at[idx_v.at[b * IDX_ROWS + k]],
                    dg_v.at[pl.ds(b * CHUNK + k * 128, 128)], sem_g)

        def wait_gather(b):
            for k in range(IDX_ROWS):
                pltpu.make_async_copy(
                    d_hbm.at[idx_v.at[b * IDX_ROWS + k]],
                    dg_v.at[pl.ds(b * CHUNK + k * 128, 128)], sem_g).wait()

        def out_desc(j, b):
            return pltpu.make_async_copy(
                dg_v.at[pl.ds(b * CHUNK, CHUNK)],
                out_hbm.at[pl.ds(cid_of(j) * CHUNK, CHUNK)], sem_o)

        issue_in(0, 0)
        wait_in(0)
        issue_gather(0)
        issue_in(1, 1)

        @pl.loop(0, nj)
        def _chunk(j):
            b = lax.rem(j, 2)
            nb = 1 - b

            @pl.when(j + 1 < nj)
            def _():
                wait_in(nb)

                @pl.when(j >= 1)
                def _():
                    out_desc(j - 1, nb).wait()

                issue_gather(nb)

            wait_gather(b)
            pltpu.async_copy(dg_v.at[pl.ds(b * CHUNK, CHUNK)],
                             out_hbm.at[pl.ds(cid_of(j) * CHUNK, CHUNK)],
                             sem_o)

            @pl.when(j + 2 < nj)
            def _():
                issue_in(j + 2, b)

        @pl.when(nj >= 2)
        def _():
            out_desc(nj - 2, lax.rem(nj - 2, 2)).wait()

        out_desc(nj - 1, lax.rem(nj - 1, 2)).wait()

    return sc_kernel


BT = 1280


def _tc_block(dg_ref, cos_ref, out_ref):
    d = dg_ref[0, 0, :][:, None]
    c = cos_ref[0, 0, :][:, None]
    j = lax.broadcasted_iota(jnp.int32, (1, NUM_RADIAL * NUM_SPHERICAL), 1)
    ro = (j % NUM_RADIAL).astype(jnp.float32) / (NUM_RADIAL - 1)
    so = ((j // NUM_RADIAL).astype(jnp.float32)
          * (2.0 / (NUM_SPHERICAL - 1)) - 1.0)
    dd = d - ro
    cc = c - so
    out_ref[...] = jnp.exp(R_COEFF * dd * dd + S_COEFF * cc * cc)


def _tc_outer(dg, cosphi, T):
    nb = T // BT
    return pl.pallas_call(
        _tc_block,
        grid=(nb,),
        in_specs=[pl.BlockSpec((1, 1, BT), lambda i: (i, 0, 0)),
                  pl.BlockSpec((1, 1, BT), lambda i: (i, 0, 0))],
        out_specs=pl.BlockSpec((BT, NUM_RADIAL * NUM_SPHERICAL),
                               lambda i: (i, 0)),
        out_shape=jax.ShapeDtypeStruct((T, NUM_RADIAL * NUM_SPHERICAL),
                                       jnp.float32),
    )(dg.reshape(nb, 1, BT), cosphi.reshape(nb, 1, BT))


def kernel(D_ca, cosphi_cab, id3_ca):
    T = cosphi_cab.shape[0]
    id3_2d = jnp.asarray(id3_ca, jnp.int32).reshape(T // 128, 128)
    dg = _make_sc_gather(T)(jnp.asarray(D_ca, jnp.float32), id3_2d)
    out = _tc_outer(dg, jnp.asarray(cosphi_cab, jnp.float32), T)
    return (out,)
